# baseline (speedup 1.0000x reference)
import functools

import jax
import jax.numpy as jnp
from jax import lax
from jax.experimental import pallas as pl
from jax.experimental.pallas import tpu as pltpu
from jax.experimental.pallas import tpu_sc as plsc

N = 10000
E = 320000
D = 128
NG = 64
NP = 10240
TILES = 32
CH = 128
CPT = 81
EP = TILES * CPT * CH
NBUF = 3
GRP = CPT // NBUF
RPS = NP // 16
RPA = 624
NBLK = 16
BR = NP // NBLK

_HIGH = lax.Precision.HIGHEST


@functools.cache
def _sc_mesh():
    return plsc.VectorSubcoreMesh(core_axis_name="c", subcore_axis_name="s")


def _deg_body(idx2_hbm, ones_hbm, zeros_hbm, out_hbm, acc_sh, idx_v, ones_v):
    c = lax.axis_index("c")
    s = lax.axis_index("s")
    pltpu.sync_copy(zeros_hbm.at[pl.ds(s * RPS, RPS)],
                    acc_sh.at[pl.ds(s * RPS, RPS)])
    pltpu.sync_copy(ones_hbm, ones_v)
    base = (c * 16 + s) * CPT
    pltpu.sync_copy(idx2_hbm.at[pl.ds(base, CPT)], idx_v)
    plsc.subcore_barrier()

    @pl.loop(0, CPT)
    def _(j):
        pltpu.sync_copy(ones_v, acc_sh.at[idx_v.at[j, 0]], add=True)

    plsc.subcore_barrier()
    pltpu.sync_copy(acc_sh.at[pl.ds(s * RPS, RPS)],
                    out_hbm.at[c, pl.ds(s * RPS, RPS)])


@jax.jit
def _sc_degrees(idx2, ones_nd, zeros_nd):
    return pl.kernel(
        _deg_body,
        out_type=jax.ShapeDtypeStruct((2, NP, D), jnp.float32),
        mesh=_sc_mesh(),
        scratch_types=[
            pltpu.VMEM_SHARED((NP, D), jnp.float32),
            pltpu.VMEM((CPT, 2, CH), jnp.int32),
            pltpu.VMEM((CH, D), jnp.float32),
        ],
    )(idx2, ones_nd, zeros_nd)


def _agg_body(h_hbm, idx2_hbm, zeros_hbm, out_hbm,
              acc_sh, idxs, msgs, g0, g1, g2, i0, i1):
    gsem = (g0, g1, g2)
    isem = (i0, i1)
    c = lax.axis_index("c")
    s = lax.axis_index("s")
    pltpu.sync_copy(zeros_hbm.at[pl.ds(s * RPA, RPA)],
                    acc_sh.at[pl.ds(s * RPA, RPA)])

    @pl.when(s == 15)
    def _():
        pltpu.sync_copy(zeros_hbm.at[pl.ds(16 * RPA, N - 16 * RPA)],
                        acc_sh.at[pl.ds(16 * RPA, N - 16 * RPA)])

    base = (c * 16 + s) * CPT
    pltpu.sync_copy(idx2_hbm.at[pl.ds(base, NBUF)], idxs.at[0])
    pltpu.async_copy(idx2_hbm.at[pl.ds(base + NBUF, NBUF)], idxs.at[1],
                     isem[1])
    plsc.subcore_barrier()
    for b in range(NBUF):
        pltpu.async_copy(h_hbm.at[idxs.at[0, b, 0]], msgs.at[b], gsem[b])

    @pl.loop(0, GRP - 1, step=2)
    def _(t0):
        for p in range(2):
            t = t0 + p
            slot = p
            nslot = 1 - p
            for b in range(NBUF):
                if b == 0:
                    pltpu.make_async_copy(
                        idx2_hbm.at[pl.ds(base, NBUF)], idxs.at[nslot],
                        isem[nslot]).wait()
                pltpu.make_async_copy(h_hbm.at[idxs.at[0, 0, 0]],
                                      msgs.at[b], gsem[b]).wait()
                pltpu.sync_copy(msgs.at[b], acc_sh.at[idxs.at[slot, b, 1]],
                                add=True)
                pltpu.async_copy(
                    h_hbm.at[idxs.at[nslot, b, 0]], msgs.at[b], gsem[b])

            @pl.when(t < GRP - 2)
            def _():
                pltpu.async_copy(
                    idx2_hbm.at[pl.ds(base + (t + 2) * NBUF, NBUF)],
                    idxs.at[slot], isem[slot])

    for b in range(NBUF):
        pltpu.make_async_copy(h_hbm.at[idxs.at[0, 0, 0]],
                              msgs.at[b], gsem[b]).wait()
        pltpu.sync_copy(msgs.at[b], acc_sh.at[idxs.at[0, b, 1]], add=True)

    plsc.subcore_barrier()
    pltpu.sync_copy(acc_sh.at[pl.ds(s * RPA, RPA)],
                    out_hbm.at[c, pl.ds(s * RPA, RPA)])

    @pl.when(s == 15)
    def _():
        pltpu.sync_copy(acc_sh.at[pl.ds(16 * RPA, N - 16 * RPA)],
                        out_hbm.at[c, pl.ds(16 * RPA, N - 16 * RPA)])


@jax.jit
def _sc_aggregate(h, idx2, zeros_nd):
    return pl.kernel(
        _agg_body,
        out_type=jax.ShapeDtypeStruct((2, NP, D), jnp.float32),
        mesh=_sc_mesh(),
        scratch_types=[
            pltpu.VMEM_SHARED((N, D), jnp.float32),
            pltpu.VMEM((2, NBUF, 2, CH), jnp.int32),
            pltpu.VMEM((NBUF, CH, D), jnp.float32),
            pltpu.SemaphoreType.DMA,
            pltpu.SemaphoreType.DMA,
            pltpu.SemaphoreType.DMA,
            pltpu.SemaphoreType.DMA,
            pltpu.SemaphoreType.DMA,
        ],
    )(h, idx2, zeros_nd)


def _mm1_body(x_ref, w_ref, b_ref, o_ref):
    o_ref[...] = jnp.dot(x_ref[...], w_ref[...],
                         preferred_element_type=jnp.float32,
                         precision=_HIGH) + b_ref[...]


@jax.jit
def _tc_matmul1(x_pad, w1t, b1r):
    return pl.pallas_call(
        _mm1_body,
        grid=(NBLK,),
        in_specs=[
            pl.BlockSpec((BR, D), lambda i: (i, 0)),
            pl.BlockSpec((D, D), lambda i: (0, 0)),
            pl.BlockSpec((1, D), lambda i: (0, 0)),
        ],
        out_specs=pl.BlockSpec((BR, D), lambda i: (i, 0)),
        out_shape=jax.ShapeDtypeStruct((NP, D), jnp.float32),
        compiler_params=pltpu.CompilerParams(
            dimension_semantics=("parallel",)),
    )(x_pad, w1t, b1r)


def _scale1_body(m_ref, dg_ref, o_ref, dis_ref):
    i = pl.program_id(0)
    deg = dg_ref[0][:, 0:1] + dg_ref[1][:, 0:1] + 1.0
    rows = i * BR + lax.broadcasted_iota(jnp.int32, (BR, 1), 0)
    dis = jnp.where(rows < N, lax.rsqrt(deg), 0.0)
    disf = jnp.broadcast_to(dis, (BR, D))
    o_ref[...] = disf * m_ref[...]
    dis_ref[...] = disf


@jax.jit
def _tc_scale1(m1, degp):
    return pl.pallas_call(
        _scale1_body,
        grid=(NBLK,),
        in_specs=[
            pl.BlockSpec((BR, D), lambda i: (i, 0)),
            pl.BlockSpec((2, BR, D), lambda i: (0, i, 0)),
        ],
        out_specs=[
            pl.BlockSpec((BR, D), lambda i: (i, 0)),
            pl.BlockSpec((BR, D), lambda i: (i, 0)),
        ],
        out_shape=[
            jax.ShapeDtypeStruct((NP, D), jnp.float32),
            jax.ShapeDtypeStruct((NP, D), jnp.float32),
        ],
        compiler_params=pltpu.CompilerParams(
            dimension_semantics=("parallel",)),
    )(m1, degp)


def _mm2_body(r_ref, h1_ref, dis_ref, w_ref, b_ref, o_ref):
    dis = dis_ref[...]
    t = dis * (r_ref[0] + r_ref[1] + h1_ref[...])
    t = jnp.where(dis > 0.0, t, 0.0)
    t = jnp.maximum(t, 0.0)
    h = jnp.dot(t, w_ref[...],
                preferred_element_type=jnp.float32,
                precision=_HIGH) + b_ref[...]
    o_ref[...] = dis * h


@jax.jit
def _tc_layer2(raw1, h1p, disf, w2t, b2r):
    return pl.pallas_call(
        _mm2_body,
        grid=(NBLK,),
        in_specs=[
            pl.BlockSpec((2, BR, D), lambda i: (0, i, 0)),
            pl.BlockSpec((BR, D), lambda i: (i, 0)),
            pl.BlockSpec((BR, D), lambda i: (i, 0)),
            pl.BlockSpec((D, D), lambda i: (0, 0)),
            pl.BlockSpec((1, D), lambda i: (0, 0)),
        ],
        out_specs=pl.BlockSpec((BR, D), lambda i: (i, 0)),
        out_shape=jax.ShapeDtypeStruct((NP, D), jnp.float32),
        compiler_params=pltpu.CompilerParams(
            dimension_semantics=("parallel",)),
    )(raw1, h1p, disf, w2t, b2r)


def _pool_body(r_ref, h2_ref, dis_ref, b_ref, o_ref, acc, cnt):
    i = pl.program_id(0)

    @pl.when(i == 0)
    def _():
        acc[...] = jnp.zeros((NG, D), jnp.float32)
        cnt[...] = jnp.zeros((NG, D), jnp.float32)

    dis = dis_ref[...]
    h2 = dis * (r_ref[0] + r_ref[1] + h2_ref[...])
    h2 = jnp.where(dis > 0.0, h2, 0.0)
    onehot = (b_ref[...] == lax.broadcasted_iota(jnp.int32, (BR, NG), 1))
    onehot = onehot.astype(jnp.float32)
    dn = (((0,), (0,)), ((), ()))
    acc[...] += lax.dot_general(onehot, h2, dn,
                                preferred_element_type=jnp.float32,
                                precision=_HIGH)
    cnt[...] += lax.dot_general(onehot, jnp.ones((BR, D), jnp.float32), dn,
                                preferred_element_type=jnp.float32,
                                precision=_HIGH)

    @pl.when(i == NBLK - 1)
    def _():
        o_ref[...] = acc[...] / jnp.maximum(cnt[...], 1.0)


@jax.jit
def _tc_pool(raw2, h2p, disf, batch2d):
    return pl.pallas_call(
        _pool_body,
        grid=(NBLK,),
        in_specs=[
            pl.BlockSpec((2, BR, D), lambda i: (0, i, 0)),
            pl.BlockSpec((BR, D), lambda i: (i, 0)),
            pl.BlockSpec((BR, D), lambda i: (i, 0)),
            pl.BlockSpec((BR, 1), lambda i: (i, 0)),
        ],
        out_specs=pl.BlockSpec((NG, D), lambda i: (0, 0)),
        out_shape=jax.ShapeDtypeStruct((NG, D), jnp.float32),
        scratch_shapes=[
            pltpu.VMEM((NG, D), jnp.float32),
            pltpu.VMEM((NG, D), jnp.float32),
        ],
    )(raw2, h2p, disf, batch2d)


def kernel(x, edge_index, batch, W1, b1, W2, b2):
    row = edge_index[0]
    col = edge_index[1]
    colp = jnp.concatenate(
        [col, jnp.full((EP - E,), N, jnp.int32)]).reshape(EP // CH, CH)
    rowp = jnp.concatenate(
        [row, jnp.zeros((EP - E,), jnp.int32)]).reshape(EP // CH, CH)
    idx2 = jnp.stack([colp, rowp], axis=1)
    x_pad = jnp.zeros((NP, D), jnp.float32).at[:N].set(x)
    batch2d = jnp.concatenate(
        [batch, jnp.full((NP - N,), NG, jnp.int32)]).reshape(NP, 1)
    zeros_nd = jnp.zeros((NP, D), jnp.float32)
    ones_nd = jnp.ones((CH, D), jnp.float32)
    w1t = W1.T
    w2t = W2.T
    b1r = b1.reshape(1, D)
    b2r = b2.reshape(1, D)

    degp = _sc_degrees(idx2, ones_nd, zeros_nd)
    m1 = _tc_matmul1(x_pad, w1t, b1r)
    h1p, disf = _tc_scale1(m1, degp)
    raw1 = _sc_aggregate(h1p, idx2, zeros_nd)
    h2p = _tc_layer2(raw1, h1p, disf, w2t, b2r)
    raw2 = _sc_aggregate(h2p, idx2, zeros_nd)
    return _tc_pool(raw2, h2p, disf, batch2d)


# speedup vs baseline: 3.6337x; 3.6337x over previous
import functools

import jax
import jax.numpy as jnp
from jax import lax
from jax.experimental import pallas as pl
from jax.experimental.pallas import tpu as pltpu
from jax.experimental.pallas import tpu_sc as plsc

N = 10000
E = 320000
D = 128
NG = 64
NP = 10240
TILES = 32
CH = 128
CPT = 81
EP = TILES * CPT * CH
NBUF = 3
GRP = CPT // NBUF
RPS = NP // 16
RPA = 624
NBLK = 16
BR = NP // NBLK

_HIGH = lax.Precision.HIGHEST


@functools.cache
def _sc_mesh():
    return plsc.VectorSubcoreMesh(core_axis_name="c", subcore_axis_name="s")


def _deg_body(idx2_hbm, ones_hbm, zeros_hbm, out_hbm, acc_sh, idx_v, ones_v):
    c = lax.axis_index("c")
    s = lax.axis_index("s")
    pltpu.sync_copy(zeros_hbm.at[pl.ds(s * RPS, RPS)],
                    acc_sh.at[pl.ds(s * RPS, RPS)])
    pltpu.sync_copy(ones_hbm, ones_v)
    base = (c * 16 + s) * CPT
    pltpu.sync_copy(idx2_hbm.at[pl.ds(base, CPT)], idx_v)
    plsc.subcore_barrier()

    @pl.loop(0, CPT)
    def _(j):
        pltpu.sync_copy(ones_v, acc_sh.at[idx_v.at[j, 0]], add=True)

    plsc.subcore_barrier()
    pltpu.sync_copy(acc_sh.at[pl.ds(s * RPS, RPS)],
                    out_hbm.at[c, pl.ds(s * RPS, RPS)])


@jax.jit
def _sc_degrees(idx2, ones_nd, zeros_nd):
    return pl.kernel(
        _deg_body,
        out_type=jax.ShapeDtypeStruct((2, NP, D), jnp.float32),
        mesh=_sc_mesh(),
        scratch_types=[
            pltpu.VMEM_SHARED((NP, D), jnp.float32),
            pltpu.VMEM((CPT, 2, CH), jnp.int32),
            pltpu.VMEM((CH, D), jnp.float32),
        ],
    )(idx2, ones_nd, zeros_nd)


def _agg_body(h_hbm, idx2_hbm, zeros_hbm, out_hbm,
              acc_sh, idxs, msgs, g0, g1, g2, i0, i1):
    gsem = (g0, g1, g2)
    isem = (i0, i1)
    c = lax.axis_index("c")
    s = lax.axis_index("s")
    pltpu.sync_copy(zeros_hbm.at[pl.ds(s * RPA, RPA)],
                    acc_sh.at[pl.ds(s * RPA, RPA)])

    @pl.when(s == 15)
    def _():
        pltpu.sync_copy(zeros_hbm.at[pl.ds(16 * RPA, N - 16 * RPA)],
                        acc_sh.at[pl.ds(16 * RPA, N - 16 * RPA)])

    base = (c * 16 + s) * CPT
    pltpu.sync_copy(idx2_hbm.at[pl.ds(base, NBUF)], idxs.at[0])
    pltpu.async_copy(idx2_hbm.at[pl.ds(base + NBUF, NBUF)], idxs.at[1],
                     isem[1])
    plsc.subcore_barrier()
    for b in range(NBUF):
        pltpu.async_copy(h_hbm.at[idxs.at[0, b, 0]], msgs.at[b], gsem[b])

    @pl.loop(0, GRP - 1, step=2)
    def _(t0):
        for p in range(2):
            t = t0 + p
            slot = p
            nslot = 1 - p
            for b in range(NBUF):
                if b == 0:
                    pltpu.make_async_copy(
                        idx2_hbm.at[pl.ds(base, NBUF)], idxs.at[nslot],
                        isem[nslot]).wait()
                pltpu.make_async_copy(h_hbm.at[idxs.at[0, 0, 0]],
                                      msgs.at[b], gsem[b]).wait()
                pltpu.sync_copy(msgs.at[b], acc_sh.at[idxs.at[slot, b, 1]],
                                add=True)
                pltpu.async_copy(
                    h_hbm.at[idxs.at[nslot, b, 0]], msgs.at[b], gsem[b])

            @pl.when(t < GRP - 2)
            def _():
                pltpu.async_copy(
                    idx2_hbm.at[pl.ds(base + (t + 2) * NBUF, NBUF)],
                    idxs.at[slot], isem[slot])

    for b in range(NBUF):
        pltpu.make_async_copy(h_hbm.at[idxs.at[0, 0, 0]],
                              msgs.at[b], gsem[b]).wait()
        pltpu.sync_copy(msgs.at[b], acc_sh.at[idxs.at[0, b, 1]], add=True)

    plsc.subcore_barrier()
    pltpu.sync_copy(acc_sh.at[pl.ds(s * RPA, RPA)],
                    out_hbm.at[c, pl.ds(s * RPA, RPA)])

    @pl.when(s == 15)
    def _():
        pltpu.sync_copy(acc_sh.at[pl.ds(16 * RPA, N - 16 * RPA)],
                        out_hbm.at[c, pl.ds(16 * RPA, N - 16 * RPA)])


@jax.jit
def _sc_aggregate(h, idx2, zeros_nd):
    return pl.kernel(
        _agg_body,
        out_type=jax.ShapeDtypeStruct((2, NP, D), jnp.float32),
        mesh=_sc_mesh(),
        scratch_types=[
            pltpu.VMEM_SHARED((N, D), jnp.float32),
            pltpu.VMEM((2, NBUF, 2, CH), jnp.int32),
            pltpu.VMEM((NBUF, CH, D), jnp.float32),
            pltpu.SemaphoreType.DMA,
            pltpu.SemaphoreType.DMA,
            pltpu.SemaphoreType.DMA,
            pltpu.SemaphoreType.DMA,
            pltpu.SemaphoreType.DMA,
        ],
    )(h, idx2, zeros_nd)


def _mm1_body(x_ref, w_ref, b_ref, o_ref):
    o_ref[...] = jnp.dot(x_ref[...], w_ref[...],
                         preferred_element_type=jnp.float32,
                         precision=_HIGH) + b_ref[...]


@jax.jit
def _tc_matmul1(x_pad, w1t, b1r):
    return pl.pallas_call(
        _mm1_body,
        grid=(NBLK,),
        in_specs=[
            pl.BlockSpec((BR, D), lambda i: (i, 0)),
            pl.BlockSpec((D, D), lambda i: (0, 0)),
            pl.BlockSpec((1, D), lambda i: (0, 0)),
        ],
        out_specs=pl.BlockSpec((BR, D), lambda i: (i, 0)),
        out_shape=jax.ShapeDtypeStruct((NP, D), jnp.float32),
        compiler_params=pltpu.CompilerParams(
            dimension_semantics=("parallel",)),
    )(x_pad, w1t, b1r)


def _scale1_body(m_ref, dg_ref, o_ref, dis_ref):
    i = pl.program_id(0)
    deg = dg_ref[0][:, 0:1] + dg_ref[1][:, 0:1] + 1.0
    rows = i * BR + lax.broadcasted_iota(jnp.int32, (BR, 1), 0)
    dis = jnp.where(rows < N, lax.rsqrt(deg), 0.0)
    disf = jnp.broadcast_to(dis, (BR, D))
    o_ref[...] = disf * m_ref[...]
    dis_ref[...] = disf


@jax.jit
def _tc_scale1(m1, degp):
    return pl.pallas_call(
        _scale1_body,
        grid=(NBLK,),
        in_specs=[
            pl.BlockSpec((BR, D), lambda i: (i, 0)),
            pl.BlockSpec((2, BR, D), lambda i: (0, i, 0)),
        ],
        out_specs=[
            pl.BlockSpec((BR, D), lambda i: (i, 0)),
            pl.BlockSpec((BR, D), lambda i: (i, 0)),
        ],
        out_shape=[
            jax.ShapeDtypeStruct((NP, D), jnp.float32),
            jax.ShapeDtypeStruct((NP, D), jnp.float32),
        ],
        compiler_params=pltpu.CompilerParams(
            dimension_semantics=("parallel",)),
    )(m1, degp)


def _mm2_body(r_ref, h1_ref, dis_ref, w_ref, b_ref, o_ref):
    dis = dis_ref[...]
    t = dis * (r_ref[0] + r_ref[1] + h1_ref[...])
    t = jnp.where(dis > 0.0, t, 0.0)
    t = jnp.maximum(t, 0.0)
    h = jnp.dot(t, w_ref[...],
                preferred_element_type=jnp.float32,
                precision=_HIGH) + b_ref[...]
    o_ref[...] = dis * h


@jax.jit
def _tc_layer2(raw1, h1p, disf, w2t, b2r):
    return pl.pallas_call(
        _mm2_body,
        grid=(NBLK,),
        in_specs=[
            pl.BlockSpec((2, BR, D), lambda i: (0, i, 0)),
            pl.BlockSpec((BR, D), lambda i: (i, 0)),
            pl.BlockSpec((BR, D), lambda i: (i, 0)),
            pl.BlockSpec((D, D), lambda i: (0, 0)),
            pl.BlockSpec((1, D), lambda i: (0, 0)),
        ],
        out_specs=pl.BlockSpec((BR, D), lambda i: (i, 0)),
        out_shape=jax.ShapeDtypeStruct((NP, D), jnp.float32),
        compiler_params=pltpu.CompilerParams(
            dimension_semantics=("parallel",)),
    )(raw1, h1p, disf, w2t, b2r)


def _pool_body(r_ref, h2_ref, dis_ref, b_ref, o_ref, acc, cnt):
    i = pl.program_id(0)

    @pl.when(i == 0)
    def _():
        acc[...] = jnp.zeros((NG, D), jnp.float32)
        cnt[...] = jnp.zeros((NG, D), jnp.float32)

    dis = dis_ref[...]
    h2 = dis * (r_ref[0] + r_ref[1] + h2_ref[...])
    h2 = jnp.where(dis > 0.0, h2, 0.0)
    onehot = (b_ref[...] == lax.broadcasted_iota(jnp.int32, (BR, NG), 1))
    onehot = onehot.astype(jnp.float32)
    dn = (((0,), (0,)), ((), ()))
    acc[...] += lax.dot_general(onehot, h2, dn,
                                preferred_element_type=jnp.float32,
                                precision=_HIGH)
    cnt[...] += lax.dot_general(onehot, jnp.ones((BR, D), jnp.float32), dn,
                                preferred_element_type=jnp.float32,
                                precision=_HIGH)

    @pl.when(i == NBLK - 1)
    def _():
        o_ref[...] = acc[...] / jnp.maximum(cnt[...], 1.0)


@jax.jit
def _tc_pool(raw2, h2p, disf, batch2d):
    return pl.pallas_call(
        _pool_body,
        grid=(NBLK,),
        in_specs=[
            pl.BlockSpec((2, BR, D), lambda i: (0, i, 0)),
            pl.BlockSpec((BR, D), lambda i: (i, 0)),
            pl.BlockSpec((BR, D), lambda i: (i, 0)),
            pl.BlockSpec((BR, 1), lambda i: (i, 0)),
        ],
        out_specs=pl.BlockSpec((NG, D), lambda i: (0, 0)),
        out_shape=jax.ShapeDtypeStruct((NG, D), jnp.float32),
        scratch_shapes=[
            pltpu.VMEM((NG, D), jnp.float32),
            pltpu.VMEM((NG, D), jnp.float32),
        ],
    )(raw2, h2p, disf, batch2d)


def kernel(x, edge_index, batch, W1, b1, W2, b2):
    row = edge_index[0]
    col = edge_index[1]
    padk = jnp.arange(EP - E, dtype=jnp.int32)
    colp = jnp.concatenate([col, N + padk % (NP - N)]).reshape(EP // CH, CH)
    rowp = jnp.concatenate([row, padk % N]).reshape(EP // CH, CH)
    idx2 = jnp.stack([colp, rowp], axis=1)
    x_pad = jnp.zeros((NP, D), jnp.float32).at[:N].set(x)
    batch2d = jnp.concatenate(
        [batch, jnp.full((NP - N,), NG, jnp.int32)]).reshape(NP, 1)
    zeros_nd = jnp.zeros((NP, D), jnp.float32)
    ones_nd = jnp.ones((CH, D), jnp.float32)
    w1t = W1.T
    w2t = W2.T
    b1r = b1.reshape(1, D)
    b2r = b2.reshape(1, D)

    degp = _sc_degrees(idx2, ones_nd, zeros_nd)
    m1 = _tc_matmul1(x_pad, w1t, b1r)
    h1p, disf = _tc_scale1(m1, degp)
    raw1 = _sc_aggregate(h1p, idx2, zeros_nd)
    h2p = _tc_layer2(raw1, h1p, disf, w2t, b2r)
    raw2 = _sc_aggregate(h2p, idx2, zeros_nd)
    return _tc_pool(raw2, h2p, disf, batch2d)


# speedup vs baseline: 4.3288x; 1.1913x over previous
import dataclasses
import functools

import jax
import jax.numpy as jnp
from jax import lax
from jax.experimental import pallas as pl
from jax.experimental.pallas import tpu as pltpu
from jax.experimental.pallas import tpu_sc as plsc

N = 10000
E = 320000
D = 128
NG = 64
NP = 10240
TILES = 32
CH = 128
CPT = 81
EP = TILES * CPT * CH
NBUF = 3
GRP = CPT // NBUF
RPS = NP // 16
RPA = 624
NBLK = 16
BR = NP // NBLK

_HIGH = lax.Precision.HIGHEST


@functools.cache
def _sc_mesh():
    return plsc.VectorSubcoreMesh(core_axis_name="c", subcore_axis_name="s")


def _deg_body(idx2_hbm, out_hbm, hist_v, idx_v):
    c = lax.axis_index("c")
    s = lax.axis_index("s")
    base = (c * 16 + s) * CPT
    pltpu.sync_copy(idx2_hbm.at[pl.ds(base, CPT)], idx_v)

    @pl.loop(0, NP, step=16)
    def _(i):
        hist_v[pl.ds(i, 16)] = jnp.zeros((16,), jnp.float32)

    ones16 = jnp.ones((16,), jnp.float32)

    @pl.loop(0, CPT)
    def _(j):
        @pl.loop(0, CH, step=16)
        def _(k):
            idx16 = idx_v[j, 0, pl.ds(k, 16)]
            plsc.addupdate_scatter(hist_v, [idx16], ones16)

    pltpu.sync_copy(hist_v, out_hbm.at[c * 16 + s])


@jax.jit
def _sc_degrees(idx2):
    cp = pltpu.CompilerParams()
    if "needs_layout_passes" in pltpu.CompilerParams.__dataclass_fields__:
        cp = dataclasses.replace(cp, needs_layout_passes=False)
    return pl.kernel(
        _deg_body,
        out_type=jax.ShapeDtypeStruct((TILES, NP), jnp.float32),
        mesh=_sc_mesh(),
        compiler_params=cp,
        scratch_types=[
            pltpu.VMEM((NP,), jnp.float32),
            pltpu.VMEM((CPT, 2, CH), jnp.int32),
        ],
    )(idx2)


def _agg_body(h_hbm, idx2_hbm, zeros_hbm, out_hbm,
              acc_sh, idxs, msgs, g0, g1, g2, i0, i1):
    gsem = (g0, g1, g2)
    isem = (i0, i1)
    c = lax.axis_index("c")
    s = lax.axis_index("s")
    pltpu.sync_copy(zeros_hbm.at[pl.ds(s * RPA, RPA)],
                    acc_sh.at[pl.ds(s * RPA, RPA)])

    @pl.when(s == 15)
    def _():
        pltpu.sync_copy(zeros_hbm.at[pl.ds(16 * RPA, N - 16 * RPA)],
                        acc_sh.at[pl.ds(16 * RPA, N - 16 * RPA)])

    base = (c * 16 + s) * CPT
    pltpu.sync_copy(idx2_hbm.at[pl.ds(base, NBUF)], idxs.at[0])
    pltpu.async_copy(idx2_hbm.at[pl.ds(base + NBUF, NBUF)], idxs.at[1],
                     isem[1])
    plsc.subcore_barrier()
    for b in range(NBUF):
        pltpu.async_copy(h_hbm.at[idxs.at[0, b, 0]], msgs.at[b], gsem[b])

    @pl.loop(0, GRP - 1, step=2)
    def _(t0):
        for p in range(2):
            t = t0 + p
            slot = p
            nslot = 1 - p
            for b in range(NBUF):
                if b == 0:
                    pltpu.make_async_copy(
                        idx2_hbm.at[pl.ds(base, NBUF)], idxs.at[nslot],
                        isem[nslot]).wait()
                pltpu.make_async_copy(h_hbm.at[idxs.at[0, 0, 0]],
                                      msgs.at[b], gsem[b]).wait()
                pltpu.sync_copy(msgs.at[b], acc_sh.at[idxs.at[slot, b, 1]],
                                add=True)
                pltpu.async_copy(
                    h_hbm.at[idxs.at[nslot, b, 0]], msgs.at[b], gsem[b])

            @pl.when(t < GRP - 2)
            def _():
                pltpu.async_copy(
                    idx2_hbm.at[pl.ds(base + (t + 2) * NBUF, NBUF)],
                    idxs.at[slot], isem[slot])

    for b in range(NBUF):
        pltpu.make_async_copy(h_hbm.at[idxs.at[0, 0, 0]],
                              msgs.at[b], gsem[b]).wait()
        pltpu.sync_copy(msgs.at[b], acc_sh.at[idxs.at[0, b, 1]], add=True)

    plsc.subcore_barrier()
    pltpu.sync_copy(acc_sh.at[pl.ds(s * RPA, RPA)],
                    out_hbm.at[c, pl.ds(s * RPA, RPA)])

    @pl.when(s == 15)
    def _():
        pltpu.sync_copy(acc_sh.at[pl.ds(16 * RPA, N - 16 * RPA)],
                        out_hbm.at[c, pl.ds(16 * RPA, N - 16 * RPA)])


@jax.jit
def _sc_aggregate(h, idx2, zeros_nd):
    return pl.kernel(
        _agg_body,
        out_type=jax.ShapeDtypeStruct((2, NP, D), jnp.float32),
        mesh=_sc_mesh(),
        scratch_types=[
            pltpu.VMEM_SHARED((N, D), jnp.float32),
            pltpu.VMEM((2, NBUF, 2, CH), jnp.int32),
            pltpu.VMEM((NBUF, CH, D), jnp.float32),
            pltpu.SemaphoreType.DMA,
            pltpu.SemaphoreType.DMA,
            pltpu.SemaphoreType.DMA,
            pltpu.SemaphoreType.DMA,
            pltpu.SemaphoreType.DMA,
        ],
    )(h, idx2, zeros_nd)


def _mm1_body(x_ref, w_ref, b_ref, o_ref):
    o_ref[...] = jnp.dot(x_ref[...], w_ref[...],
                         preferred_element_type=jnp.float32,
                         precision=_HIGH) + b_ref[...]


@jax.jit
def _tc_matmul1(x_pad, w1t, b1r):
    return pl.pallas_call(
        _mm1_body,
        grid=(NBLK,),
        in_specs=[
            pl.BlockSpec((BR, D), lambda i: (i, 0)),
            pl.BlockSpec((D, D), lambda i: (0, 0)),
            pl.BlockSpec((1, D), lambda i: (0, 0)),
        ],
        out_specs=pl.BlockSpec((BR, D), lambda i: (i, 0)),
        out_shape=jax.ShapeDtypeStruct((NP, D), jnp.float32),
        compiler_params=pltpu.CompilerParams(
            dimension_semantics=("parallel",)),
    )(x_pad, w1t, b1r)


def _scale1_body(m_ref, dg_ref, o_ref, dis_ref):
    i = pl.program_id(0)
    deg = jnp.sum(dg_ref[...], axis=0) + 1.0
    dis_row = lax.rsqrt(deg).reshape(BR, 1)
    rows = i * BR + lax.broadcasted_iota(jnp.int32, (BR, 1), 0)
    dis = jnp.where(rows < N, dis_row, 0.0)
    disf = jnp.broadcast_to(dis, (BR, D))
    o_ref[...] = disf * m_ref[...]
    dis_ref[...] = disf


@jax.jit
def _tc_scale1(m1, degp):
    return pl.pallas_call(
        _scale1_body,
        grid=(NBLK,),
        in_specs=[
            pl.BlockSpec((BR, D), lambda i: (i, 0)),
            pl.BlockSpec((TILES, BR), lambda i: (0, i)),
        ],
        out_specs=[
            pl.BlockSpec((BR, D), lambda i: (i, 0)),
            pl.BlockSpec((BR, D), lambda i: (i, 0)),
        ],
        out_shape=[
            jax.ShapeDtypeStruct((NP, D), jnp.float32),
            jax.ShapeDtypeStruct((NP, D), jnp.float32),
        ],
        compiler_params=pltpu.CompilerParams(
            dimension_semantics=("parallel",)),
    )(m1, degp)


def _mm2_body(r_ref, h1_ref, dis_ref, w_ref, b_ref, o_ref):
    dis = dis_ref[...]
    t = dis * (r_ref[0] + r_ref[1] + h1_ref[...])
    t = jnp.where(dis > 0.0, t, 0.0)
    t = jnp.maximum(t, 0.0)
    h = jnp.dot(t, w_ref[...],
                preferred_element_type=jnp.float32,
                precision=_HIGH) + b_ref[...]
    o_ref[...] = dis * h


@jax.jit
def _tc_layer2(raw1, h1p, disf, w2t, b2r):
    return pl.pallas_call(
        _mm2_body,
        grid=(NBLK,),
        in_specs=[
            pl.BlockSpec((2, BR, D), lambda i: (0, i, 0)),
            pl.BlockSpec((BR, D), lambda i: (i, 0)),
            pl.BlockSpec((BR, D), lambda i: (i, 0)),
            pl.BlockSpec((D, D), lambda i: (0, 0)),
            pl.BlockSpec((1, D), lambda i: (0, 0)),
        ],
        out_specs=pl.BlockSpec((BR, D), lambda i: (i, 0)),
        out_shape=jax.ShapeDtypeStruct((NP, D), jnp.float32),
        compiler_params=pltpu.CompilerParams(
            dimension_semantics=("parallel",)),
    )(raw1, h1p, disf, w2t, b2r)


def _pool_body(r_ref, h2_ref, dis_ref, b_ref, o_ref, acc, cnt):
    i = pl.program_id(0)

    @pl.when(i == 0)
    def _():
        acc[...] = jnp.zeros((NG, D), jnp.float32)
        cnt[...] = jnp.zeros((NG, D), jnp.float32)

    dis = dis_ref[...]
    h2 = dis * (r_ref[0] + r_ref[1] + h2_ref[...])
    h2 = jnp.where(dis > 0.0, h2, 0.0)
    onehot = (b_ref[...] == lax.broadcasted_iota(jnp.int32, (BR, NG), 1))
    onehot = onehot.astype(jnp.float32)
    dn = (((0,), (0,)), ((), ()))
    acc[...] += lax.dot_general(onehot, h2, dn,
                                preferred_element_type=jnp.float32,
                                precision=_HIGH)
    cnt[...] += lax.dot_general(onehot, jnp.ones((BR, D), jnp.float32), dn,
                                preferred_element_type=jnp.float32,
                                precision=_HIGH)

    @pl.when(i == NBLK - 1)
    def _():
        o_ref[...] = acc[...] / jnp.maximum(cnt[...], 1.0)


@jax.jit
def _tc_pool(raw2, h2p, disf, batch2d):
    return pl.pallas_call(
        _pool_body,
        grid=(NBLK,),
        in_specs=[
            pl.BlockSpec((2, BR, D), lambda i: (0, i, 0)),
            pl.BlockSpec((BR, D), lambda i: (i, 0)),
            pl.BlockSpec((BR, D), lambda i: (i, 0)),
            pl.BlockSpec((BR, 1), lambda i: (i, 0)),
        ],
        out_specs=pl.BlockSpec((NG, D), lambda i: (0, 0)),
        out_shape=jax.ShapeDtypeStruct((NG, D), jnp.float32),
        scratch_shapes=[
            pltpu.VMEM((NG, D), jnp.float32),
            pltpu.VMEM((NG, D), jnp.float32),
        ],
    )(raw2, h2p, disf, batch2d)


def kernel(x, edge_index, batch, W1, b1, W2, b2):
    row = edge_index[0]
    col = edge_index[1]
    padk = jnp.arange(EP - E, dtype=jnp.int32)
    colp = jnp.concatenate([col, N + padk % (NP - N)]).reshape(EP // CH, CH)
    rowp = jnp.concatenate([row, padk % N]).reshape(EP // CH, CH)
    idx2 = jnp.stack([colp, rowp], axis=1)
    x_pad = jnp.zeros((NP, D), jnp.float32).at[:N].set(x)
    batch2d = jnp.concatenate(
        [batch, jnp.full((NP - N,), NG, jnp.int32)]).reshape(NP, 1)
    zeros_nd = jnp.zeros((NP, D), jnp.float32)
    w1t = W1.T
    w2t = W2.T
    b1r = b1.reshape(1, D)
    b2r = b2.reshape(1, D)

    degp = _sc_degrees(idx2)
    m1 = _tc_matmul1(x_pad, w1t, b1r)
    h1p, disf = _tc_scale1(m1, degp)
    raw1 = _sc_aggregate(h1p, idx2, zeros_nd)
    h2p = _tc_layer2(raw1, h1p, disf, w2t, b2r)
    raw2 = _sc_aggregate(h2p, idx2, zeros_nd)
    return _tc_pool(raw2, h2p, disf, batch2d)


# speedup vs baseline: 4.7737x; 1.1028x over previous
import dataclasses
import functools

import jax
import jax.numpy as jnp
from jax import lax
from jax.experimental import pallas as pl
from jax.experimental.pallas import tpu as pltpu
from jax.experimental.pallas import tpu_sc as plsc

N = 10000
E = 320000
D = 128
NG = 64
NP = 10240
TILES = 32
CH = 128
CPT = 81
EP = TILES * CPT * CH
NBUF = 3
GRP = CPT // NBUF
RPS = NP // 16
RPA = 624
NBLK = 16
BR = NP // NBLK

_HIGH = lax.Precision.HIGHEST


@functools.cache
def _sc_mesh():
    return plsc.VectorSubcoreMesh(core_axis_name="c", subcore_axis_name="s")


def _deg_body(idx2_hbm, out_hbm, hist_v, idx_v):
    c = lax.axis_index("c")
    s = lax.axis_index("s")
    base = (c * 16 + s) * CPT
    pltpu.sync_copy(idx2_hbm.at[pl.ds(base, CPT)], idx_v)

    @pl.loop(0, NP, step=16)
    def _(i):
        hist_v[pl.ds(i, 16)] = jnp.zeros((16,), jnp.float32)

    ones16 = jnp.ones((16,), jnp.float32)

    @pl.loop(0, CPT)
    def _(j):
        @pl.loop(0, CH, step=16)
        def _(k):
            idx16 = idx_v[j, 0, pl.ds(k, 16)]
            plsc.addupdate_scatter(hist_v, [idx16], ones16)

    pltpu.sync_copy(hist_v, out_hbm.at[c * 16 + s])


@jax.jit
def _sc_degrees(idx2):
    cp = pltpu.CompilerParams()
    if "needs_layout_passes" in pltpu.CompilerParams.__dataclass_fields__:
        cp = dataclasses.replace(cp, needs_layout_passes=False)
    return pl.kernel(
        _deg_body,
        out_type=jax.ShapeDtypeStruct((TILES, NP), jnp.float32),
        mesh=_sc_mesh(),
        compiler_params=cp,
        scratch_types=[
            pltpu.VMEM((NP,), jnp.float32),
            pltpu.VMEM((CPT, 2, CH), jnp.int32),
        ],
    )(idx2)


IRING = 6


def _agg_body(h_hbm, idx2_hbm, zeros_hbm, out_hbm,
              acc_sh, idxs, msgs, g0, g1, g2, s0, s1, s2,
              i0, i1, i2, i3, i4, i5):
    gsem = (g0, g1, g2)
    ssem = (s0, s1, s2)
    isem = (i0, i1, i2, i3, i4, i5)
    c = lax.axis_index("c")
    s = lax.axis_index("s")
    base = (c * 16 + s) * CPT

    def wait_i(k6):
        pltpu.make_async_copy(idx2_hbm.at[base], idxs.at[k6],
                              isem[k6]).wait()

    def wait_g(b3):
        pltpu.make_async_copy(h_hbm.at[idxs.at[0, 0]], msgs.at[b3],
                              gsem[b3]).wait()

    def wait_s(b3):
        pltpu.make_async_copy(msgs.at[b3], acc_sh.at[idxs.at[0, 1]],
                              ssem[b3]).wait()

    for k in range(IRING - 1):
        pltpu.async_copy(idx2_hbm.at[base + k], idxs.at[k], isem[k])
    pltpu.sync_copy(zeros_hbm.at[pl.ds(s * RPA, RPA)],
                    acc_sh.at[pl.ds(s * RPA, RPA)])

    @pl.when(s == 15)
    def _():
        pltpu.sync_copy(zeros_hbm.at[pl.ds(16 * RPA, N - 16 * RPA)],
                        acc_sh.at[pl.ds(16 * RPA, N - 16 * RPA)])

    for b in range(2):
        wait_i(b)
        pltpu.async_copy(h_hbm.at[idxs.at[b, 0]], msgs.at[b], gsem[b])
    plsc.subcore_barrier()

    def step(j, m3, m6, wait_prev, gather, prefetch):
        wait_g(m3)
        pltpu.async_copy(msgs.at[m3], acc_sh.at[idxs.at[m6, 1]],
                         ssem[m3])
        if wait_prev:
            wait_s((m3 + 2) % NBUF)
        if gather:
            nb = (m3 + 2) % NBUF
            ni = (m6 + 2) % IRING
            wait_i(ni)
            pltpu.async_copy(h_hbm.at[idxs.at[ni, 0]], msgs.at[nb],
                             gsem[nb])
        if prefetch:
            pi = (m6 + IRING - 1) % IRING
            pltpu.async_copy(idx2_hbm.at[base + j + IRING - 1],
                             idxs.at[pi], isem[pi])

    step(0, 0, 0, False, True, True)

    @pl.loop(1, 73, step=IRING)
    def _(t):
        for p in range(IRING):
            step(t + p, (1 + p) % NBUF, (1 + p) % IRING, True, True, True)

    for j in range(73, CPT):
        step(j, j % NBUF, j % IRING, True, j + 2 <= CPT - 1,
             j + IRING - 1 <= CPT - 1)
    wait_s((CPT - 1) % NBUF)

    plsc.subcore_barrier()
    pltpu.sync_copy(acc_sh.at[pl.ds(s * RPA, RPA)],
                    out_hbm.at[c, pl.ds(s * RPA, RPA)])

    @pl.when(s == 15)
    def _():
        pltpu.sync_copy(acc_sh.at[pl.ds(16 * RPA, N - 16 * RPA)],
                        out_hbm.at[c, pl.ds(16 * RPA, N - 16 * RPA)])


@jax.jit
def _sc_aggregate(h, idx2, zeros_nd):
    return pl.kernel(
        _agg_body,
        out_type=jax.ShapeDtypeStruct((2, NP, D), jnp.float32),
        mesh=_sc_mesh(),
        scratch_types=[
            pltpu.VMEM_SHARED((N, D), jnp.float32),
            pltpu.VMEM((IRING, 2, CH), jnp.int32),
            pltpu.VMEM((NBUF, CH, D), jnp.float32),
        ] + [pltpu.SemaphoreType.DMA] * 12,
    )(h, idx2, zeros_nd)


def _mm1_body(x_ref, w_ref, b_ref, o_ref):
    o_ref[...] = jnp.dot(x_ref[...], w_ref[...],
                         preferred_element_type=jnp.float32,
                         precision=_HIGH) + b_ref[...]


@jax.jit
def _tc_matmul1(x_pad, w1t, b1r):
    return pl.pallas_call(
        _mm1_body,
        grid=(NBLK,),
        in_specs=[
            pl.BlockSpec((BR, D), lambda i: (i, 0)),
            pl.BlockSpec((D, D), lambda i: (0, 0)),
            pl.BlockSpec((1, D), lambda i: (0, 0)),
        ],
        out_specs=pl.BlockSpec((BR, D), lambda i: (i, 0)),
        out_shape=jax.ShapeDtypeStruct((NP, D), jnp.float32),
        compiler_params=pltpu.CompilerParams(
            dimension_semantics=("parallel",)),
    )(x_pad, w1t, b1r)


def _scale1_body(m_ref, dg_ref, o_ref, dis_ref):
    i = pl.program_id(0)
    deg = jnp.sum(dg_ref[...], axis=0) + 1.0
    dis_row = lax.rsqrt(deg).reshape(BR, 1)
    rows = i * BR + lax.broadcasted_iota(jnp.int32, (BR, 1), 0)
    dis = jnp.where(rows < N, dis_row, 0.0)
    disf = jnp.broadcast_to(dis, (BR, D))
    o_ref[...] = disf * m_ref[...]
    dis_ref[...] = disf


@jax.jit
def _tc_scale1(m1, degp):
    return pl.pallas_call(
        _scale1_body,
        grid=(NBLK,),
        in_specs=[
            pl.BlockSpec((BR, D), lambda i: (i, 0)),
            pl.BlockSpec((TILES, BR), lambda i: (0, i)),
        ],
        out_specs=[
            pl.BlockSpec((BR, D), lambda i: (i, 0)),
            pl.BlockSpec((BR, D), lambda i: (i, 0)),
        ],
        out_shape=[
            jax.ShapeDtypeStruct((NP, D), jnp.float32),
            jax.ShapeDtypeStruct((NP, D), jnp.float32),
        ],
        compiler_params=pltpu.CompilerParams(
            dimension_semantics=("parallel",)),
    )(m1, degp)


def _mm2_body(r_ref, h1_ref, dis_ref, w_ref, b_ref, o_ref):
    dis = dis_ref[...]
    t = dis * (r_ref[0] + r_ref[1] + h1_ref[...])
    t = jnp.where(dis > 0.0, t, 0.0)
    t = jnp.maximum(t, 0.0)
    h = jnp.dot(t, w_ref[...],
                preferred_element_type=jnp.float32,
                precision=_HIGH) + b_ref[...]
    o_ref[...] = dis * h


@jax.jit
def _tc_layer2(raw1, h1p, disf, w2t, b2r):
    return pl.pallas_call(
        _mm2_body,
        grid=(NBLK,),
        in_specs=[
            pl.BlockSpec((2, BR, D), lambda i: (0, i, 0)),
            pl.BlockSpec((BR, D), lambda i: (i, 0)),
            pl.BlockSpec((BR, D), lambda i: (i, 0)),
            pl.BlockSpec((D, D), lambda i: (0, 0)),
            pl.BlockSpec((1, D), lambda i: (0, 0)),
        ],
        out_specs=pl.BlockSpec((BR, D), lambda i: (i, 0)),
        out_shape=jax.ShapeDtypeStruct((NP, D), jnp.float32),
        compiler_params=pltpu.CompilerParams(
            dimension_semantics=("parallel",)),
    )(raw1, h1p, disf, w2t, b2r)


def _pool_body(r_ref, h2_ref, dis_ref, b_ref, o_ref, acc, cnt):
    i = pl.program_id(0)

    @pl.when(i == 0)
    def _():
        acc[...] = jnp.zeros((NG, D), jnp.float32)
        cnt[...] = jnp.zeros((NG, D), jnp.float32)

    dis = dis_ref[...]
    h2 = dis * (r_ref[0] + r_ref[1] + h2_ref[...])
    h2 = jnp.where(dis > 0.0, h2, 0.0)
    onehot = (b_ref[...] == lax.broadcasted_iota(jnp.int32, (BR, NG), 1))
    onehot = onehot.astype(jnp.float32)
    dn = (((0,), (0,)), ((), ()))
    acc[...] += lax.dot_general(onehot, h2, dn,
                                preferred_element_type=jnp.float32,
                                precision=_HIGH)
    cnt[...] += lax.dot_general(onehot, jnp.ones((BR, D), jnp.float32), dn,
                                preferred_element_type=jnp.float32,
                                precision=_HIGH)

    @pl.when(i == NBLK - 1)
    def _():
        o_ref[...] = acc[...] / jnp.maximum(cnt[...], 1.0)


@jax.jit
def _tc_pool(raw2, h2p, disf, batch2d):
    return pl.pallas_call(
        _pool_body,
        grid=(NBLK,),
        in_specs=[
            pl.BlockSpec((2, BR, D), lambda i: (0, i, 0)),
            pl.BlockSpec((BR, D), lambda i: (i, 0)),
            pl.BlockSpec((BR, D), lambda i: (i, 0)),
            pl.BlockSpec((BR, 1), lambda i: (i, 0)),
        ],
        out_specs=pl.BlockSpec((NG, D), lambda i: (0, 0)),
        out_shape=jax.ShapeDtypeStruct((NG, D), jnp.float32),
        scratch_shapes=[
            pltpu.VMEM((NG, D), jnp.float32),
            pltpu.VMEM((NG, D), jnp.float32),
        ],
    )(raw2, h2p, disf, batch2d)


def kernel(x, edge_index, batch, W1, b1, W2, b2):
    row = edge_index[0]
    col = edge_index[1]
    padk = jnp.arange(EP - E, dtype=jnp.int32)
    colp = jnp.concatenate([col, N + padk % (NP - N)]).reshape(EP // CH, CH)
    rowp = jnp.concatenate([row, padk % N]).reshape(EP // CH, CH)
    idx2 = jnp.stack([colp, rowp], axis=1)
    x_pad = jnp.zeros((NP, D), jnp.float32).at[:N].set(x)
    batch2d = jnp.concatenate(
        [batch, jnp.full((NP - N,), NG, jnp.int32)]).reshape(NP, 1)
    zeros_nd = jnp.zeros((NP, D), jnp.float32)
    w1t = W1.T
    w2t = W2.T
    b1r = b1.reshape(1, D)
    b2r = b2.reshape(1, D)

    degp = _sc_degrees(idx2)
    m1 = _tc_matmul1(x_pad, w1t, b1r)
    h1p, disf = _tc_scale1(m1, degp)
    raw1 = _sc_aggregate(h1p, idx2, zeros_nd)
    h2p = _tc_layer2(raw1, h1p, disf, w2t, b2r)
    raw2 = _sc_aggregate(h2p, idx2, zeros_nd)
    return _tc_pool(raw2, h2p, disf, batch2d)
